# baseline (speedup 1.0000x reference)
import functools

import jax
import jax.numpy as jnp
from jax import lax
from jax.experimental import pallas as pl
from jax.experimental.pallas import tpu as pltpu
from jax.experimental.pallas import tpu_sc as plsc

NUM_CORES = 2
NUM_SUBCORES = 16
NUM_WORKERS = NUM_CORES * NUM_SUBCORES
CHUNK = 16
NBUF = 2


@functools.partial(jax.jit, static_argnames=("total", "embed_dim"))
def _gather(idx_flat, table, *, total, embed_dim):
    per_w = total // NUM_WORKERS
    num_chunks = per_w // CHUNK
    assert num_chunks % NBUF == 0 and num_chunks >= 2 * NBUF
    mesh = plsc.VectorSubcoreMesh(core_axis_name="c", subcore_axis_name="s")

    @functools.partial(
        pl.kernel,
        out_type=jax.ShapeDtypeStruct((total, embed_dim), jnp.float32),
        mesh=mesh,
        scratch_types=[
            pltpu.VMEM((per_w,), jnp.int32),
            [pltpu.VMEM((CHUNK, embed_dim), jnp.float32) for _ in range(NBUF)],
            pltpu.VMEM_SHARED((NUM_SUBCORES * NBUF * CHUNK, embed_dim),
                              jnp.float32),
            [pltpu.SemaphoreType.DMA for _ in range(NBUF)],
            [pltpu.SemaphoreType.DMA for _ in range(NBUF)],
            [pltpu.SemaphoreType.DMA for _ in range(NBUF)],
        ],
    )
    def k(idx_hbm, table_hbm, out_hbm, idx_v, rows, stage, gsem, csem, wsem):
        wid = lax.axis_index("s") * NUM_CORES + lax.axis_index("c")
        sid = lax.axis_index("s")
        base = wid * per_w
        pltpu.sync_copy(idx_hbm.at[pl.ds(base, per_w)], idx_v)

        def slot(b):
            return (sid * NBUF + b) * CHUNK

        def start_gather(i, b):
            pltpu.async_copy(
                table_hbm.at[idx_v.at[pl.ds(i * CHUNK, CHUNK)]], rows[b], gsem[b]
            )

        def wait_gather(b):
            pltpu.make_async_copy(table_hbm.at[pl.ds(0, CHUNK)], rows[b],
                                  gsem[b]).wait()

        def start_stage(b):
            pltpu.async_copy(rows[b], stage.at[pl.ds(slot(b), CHUNK)], csem[b])

        def wait_stage(b):
            pltpu.make_async_copy(rows[b], stage.at[pl.ds(slot(b), CHUNK)],
                                  csem[b]).wait()

        def start_put(i, b):
            pltpu.async_copy(stage.at[pl.ds(slot(b), CHUNK)],
                             out_hbm.at[pl.ds(base + i * CHUNK, CHUNK)], wsem[b])

        def wait_put(b):
            pltpu.make_async_copy(stage.at[pl.ds(slot(b), CHUNK)],
                                  out_hbm.at[pl.ds(base, CHUNK)], wsem[b]).wait()

        for b in range(NBUF):
            start_gather(b, b)

        def body(g, _):
            for b in range(NBUF):
                i = g + b
                wait_gather(b)
                start_stage(b)
                wait_stage(b)
                start_gather(i + NBUF, b)
                start_put(i, b)
                wait_put(b)
            return 0

        lax.fori_loop(0, (num_chunks - NBUF) // NBUF, lambda g, c: body(g * NBUF, c), 0)

        for b in range(NBUF):
            i = num_chunks - NBUF + b
            wait_gather(b)
            start_stage(b)
            wait_stage(b)
            start_put(i, b)
        for b in range(NBUF):
            wait_put(b)

    return k(idx_flat, table)


def kernel(position_ids, table):
    b, s = position_ids.shape
    _, d = table.shape
    idx_flat = position_ids.reshape(b * s).astype(jnp.int32)
    out = _gather(idx_flat, table, total=b * s, embed_dim=d)
    return out.reshape(b, s, d)


# speedup vs baseline: 1.0037x; 1.0037x over previous
import functools

import jax
import jax.numpy as jnp
from jax import lax
from jax.experimental import pallas as pl
from jax.experimental.pallas import tpu as pltpu
from jax.experimental.pallas import tpu_sc as plsc

NUM_CORES = 2
NUM_SUBCORES = 16
NUM_WORKERS = NUM_CORES * NUM_SUBCORES
CHUNK = 16
NBUF = 4


@functools.partial(jax.jit, static_argnames=("total", "embed_dim"))
def _gather(idx_flat, table, *, total, embed_dim):
    per_w = total // NUM_WORKERS
    num_chunks = per_w // CHUNK
    assert num_chunks % NBUF == 0 and num_chunks >= 2 * NBUF
    mesh = plsc.VectorSubcoreMesh(core_axis_name="c", subcore_axis_name="s")

    @functools.partial(
        pl.kernel,
        out_type=jax.ShapeDtypeStruct((total, embed_dim), jnp.float32),
        mesh=mesh,
        scratch_types=[
            pltpu.VMEM((per_w,), jnp.int32),
            [pltpu.VMEM((CHUNK, embed_dim), jnp.float32) for _ in range(NBUF)],
            [pltpu.SemaphoreType.DMA for _ in range(NBUF)],
            [pltpu.SemaphoreType.DMA for _ in range(NBUF)],
        ],
    )
    def k(idx_hbm, table_hbm, out_hbm, idx_v, rows, gsem, psem):
        wid = lax.axis_index("s") * NUM_CORES + lax.axis_index("c")
        base = wid * per_w
        pltpu.sync_copy(idx_hbm.at[pl.ds(base, per_w)], idx_v)

        def start_gather(i, b):
            pltpu.async_copy(
                table_hbm.at[idx_v.at[pl.ds(i * CHUNK, CHUNK)]], rows[b], gsem[b]
            )

        def start_put(i, b):
            pltpu.async_copy(rows[b], out_hbm.at[pl.ds(base + i * CHUNK, CHUNK)],
                             psem[b])

        def wait_gather(b):
            pltpu.make_async_copy(table_hbm.at[pl.ds(0, CHUNK)], rows[b],
                                  gsem[b]).wait()

        def wait_put(b):
            pltpu.make_async_copy(rows[b], out_hbm.at[pl.ds(base, CHUNK)],
                                  psem[b]).wait()

        for b in range(NBUF):
            start_gather(b, b)

        def body(g, _):
            for b in range(NBUF):
                i = g + b
                wait_gather(b)
                start_put(i, b)
                wait_put(b)
                start_gather(i + NBUF, b)
            return 0

        lax.fori_loop(0, (num_chunks - NBUF) // NBUF, lambda g, c: body(g * NBUF, c), 0)

        for b in range(NBUF):
            i = num_chunks - NBUF + b
            wait_gather(b)
            start_put(i, b)
        for b in range(NBUF):
            wait_put(b)

    return k(idx_flat, table)


def kernel(position_ids, table):
    b, s = position_ids.shape
    _, d = table.shape
    idx_flat = position_ids.reshape(b * s).astype(jnp.int32)
    out = _gather(idx_flat, table, total=b * s, embed_dim=d)
    return out.reshape(b, s, d)


# speedup vs baseline: 1.0090x; 1.0053x over previous
import functools

import jax
import jax.numpy as jnp
from jax import lax
from jax.experimental import pallas as pl
from jax.experimental.pallas import tpu as pltpu
from jax.experimental.pallas import tpu_sc as plsc

NUM_CORES = 2
NUM_SUBCORES = 16
NUM_WORKERS = NUM_CORES * NUM_SUBCORES
CHUNK = 8
NBUF = 8


@functools.partial(jax.jit, static_argnames=("total", "embed_dim"))
def _gather(idx_flat, table, *, total, embed_dim):
    per_w = total // NUM_WORKERS
    num_chunks = per_w // CHUNK
    assert num_chunks % NBUF == 0 and num_chunks >= 2 * NBUF
    mesh = plsc.VectorSubcoreMesh(core_axis_name="c", subcore_axis_name="s")

    @functools.partial(
        pl.kernel,
        out_type=jax.ShapeDtypeStruct((total, embed_dim), jnp.float32),
        mesh=mesh,
        scratch_types=[
            pltpu.VMEM((per_w,), jnp.int32),
            [pltpu.VMEM((CHUNK, embed_dim), jnp.float32) for _ in range(NBUF)],
            [pltpu.SemaphoreType.DMA for _ in range(NBUF)],
            [pltpu.SemaphoreType.DMA for _ in range(NBUF)],
        ],
    )
    def k(idx_hbm, table_hbm, out_hbm, idx_v, rows, gsem, psem):
        wid = lax.axis_index("s") * NUM_CORES + lax.axis_index("c")
        base = wid * per_w
        pltpu.sync_copy(idx_hbm.at[pl.ds(base, per_w)], idx_v)

        def start_gather(i, b):
            pltpu.async_copy(
                table_hbm.at[idx_v.at[pl.ds(i * CHUNK, CHUNK)]], rows[b], gsem[b]
            )

        def start_put(i, b):
            pltpu.async_copy(rows[b], out_hbm.at[pl.ds(base + i * CHUNK, CHUNK)],
                             psem[b])

        def wait_gather(b):
            pltpu.make_async_copy(table_hbm.at[pl.ds(0, CHUNK)], rows[b],
                                  gsem[b]).wait()

        def wait_put(b):
            pltpu.make_async_copy(rows[b], out_hbm.at[pl.ds(base, CHUNK)],
                                  psem[b]).wait()

        for b in range(NBUF):
            start_gather(b, b)

        def body(g, _):
            for b in range(NBUF):
                i = g + b
                wait_gather(b)
                start_put(i, b)
                wait_put(b)
                start_gather(i + NBUF, b)
            return 0

        lax.fori_loop(0, (num_chunks - NBUF) // NBUF, lambda g, c: body(g * NBUF, c), 0)

        for b in range(NBUF):
            i = num_chunks - NBUF + b
            wait_gather(b)
            start_put(i, b)
        for b in range(NBUF):
            wait_put(b)

    return k(idx_flat, table)


def kernel(position_ids, table):
    b, s = position_ids.shape
    _, d = table.shape
    idx_flat = position_ids.reshape(b * s).astype(jnp.int32)
    out = _gather(idx_flat, table, total=b * s, embed_dim=d)
    return out.reshape(b, s, d)
